# baseline (speedup 1.0000x reference)
import functools

import jax
import jax.numpy as jnp
from jax import lax
from jax.experimental import pallas as pl
from jax.experimental.pallas import tpu as pltpu
from jax.experimental.pallas import tpu_sc as plsc

NC = 2
NS = 16
LANES = 16

N = 10000
E = 320000
D = 128
H = 128

EPW = E // (NC * NS)
EB = 80
NBLK = EPW // EB
RPT = 624
TAIL = N - NS * RPT


def _sc_layer_body(node_hbm, emb_hbm, src_hbm, dst_hbm, out_hbm,
                   idx_s, idx_d, rows_v, emb_v, table_sh, sem,
                   sc_sem0, sc_sem1):
    c = lax.axis_index("c")
    s = lax.axis_index("s")
    wid = c * NS + s

    pltpu.sync_copy(node_hbm.at[pl.ds(s * RPT, RPT)],
                    table_sh.at[pl.ds(s * RPT, RPT)])

    @pl.when(s == 0)
    def _():
        pltpu.sync_copy(node_hbm.at[pl.ds(NS * RPT, TAIL)],
                        table_sh.at[pl.ds(NS * RPT, TAIL)])

    plsc.subcore_barrier()

    sc_sems = (sc_sem0, sc_sem1)

    def edge_block_slot(g, b):
        base = wid * EPW + g * EB
        pltpu.sync_copy(src_hbm.at[pl.ds(base, EB)], idx_s.at[b])
        pltpu.sync_copy(dst_hbm.at[pl.ds(base, EB)], idx_d.at[b])

        def _wait_prev():
            pltpu.make_async_copy(rows_v.at[b], table_sh.at[idx_d.at[b]],
                                  sc_sems[b]).wait()

        if isinstance(g, int):
            if g >= 2:
                _wait_prev()
        else:
            pl.when(g >= 2)(_wait_prev)

        gather = pltpu.async_copy(node_hbm.at[idx_s.at[b]], rows_v.at[b], sem)
        pltpu.sync_copy(emb_hbm.at[pl.ds(base, EB)], emb_v)
        gather.wait()

        def mul_row(i, carry2):
            for d in range(H // LANES):
                sl = pl.ds(d * LANES, LANES)
                rows_v[b, i, sl] = rows_v[b, i, sl] * emb_v[i, sl]
            return carry2

        lax.fori_loop(0, EB, mul_row, 0, unroll=False)
        pltpu.async_copy(rows_v.at[b], table_sh.at[idx_d.at[b]], sc_sems[b],
                         add=True)

    def edge_block(h, carry):
        edge_block_slot(2 * h, 0)
        edge_block_slot(2 * h + 1, 1)
        return carry

    lax.fori_loop(0, NBLK // 2, edge_block, 0, unroll=False)
    if NBLK % 2:
        edge_block_slot(NBLK - 1, 0)
    pltpu.make_async_copy(rows_v.at[0], table_sh.at[idx_d.at[0]],
                          sc_sems[0]).wait()
    pltpu.make_async_copy(rows_v.at[1], table_sh.at[idx_d.at[1]],
                          sc_sems[1]).wait()
    plsc.subcore_barrier()

    pltpu.sync_copy(table_sh.at[pl.ds(s * RPT, RPT)],
                    out_hbm.at[c, pl.ds(s * RPT, RPT)])

    @pl.when(s == 0)
    def _():
        pltpu.sync_copy(table_sh.at[pl.ds(NS * RPT, TAIL)],
                        out_hbm.at[c, pl.ds(NS * RPT, TAIL)])


@functools.cache
def _get_sc_layer():
    return pl.kernel(
        _sc_layer_body,
        out_type=jax.ShapeDtypeStruct((NC, N, H), jnp.float32),
        mesh=plsc.VectorSubcoreMesh(core_axis_name="c", subcore_axis_name="s",
                                    num_cores=NC, num_subcores=NS),
        scratch_types=[
            pltpu.VMEM((2, EB), jnp.int32),
            pltpu.VMEM((2, EB), jnp.int32),
            pltpu.VMEM((2, EB, H), jnp.float32),
            pltpu.VMEM((EB, H), jnp.float32),
            pltpu.VMEM_SHARED((N, H), jnp.float32),
            pltpu.SemaphoreType.DMA,
            pltpu.SemaphoreType.DMA,
            pltpu.SemaphoreType.DMA,
        ],
    )


def _sc_layer(*args):
    return _get_sc_layer()(*args)


def _matmul_bias_body(x_ref, w_ref, b_ref, o_ref):
    o_ref[...] = jnp.dot(x_ref[...], w_ref[...],
                         preferred_element_type=jnp.float32) + b_ref[...]


def _matmul_bias(x, w_t, b, row_blk):
    rows, k = x.shape
    cols = w_t.shape[1]
    grid = rows // row_blk
    return pl.pallas_call(
        _matmul_bias_body,
        grid=(grid,),
        in_specs=[
            pl.BlockSpec((row_blk, k), lambda i: (i, 0)),
            pl.BlockSpec((k, cols), lambda i: (0, 0)),
            pl.BlockSpec((1, cols), lambda i: (0, 0)),
        ],
        out_specs=pl.BlockSpec((row_blk, cols), lambda i: (i, 0)),
        out_shape=jax.ShapeDtypeStruct((rows, cols), jnp.float32),
    )(x, w_t, b)


def _combine_body(a_ref, b_ref, n_ref, o_ref):
    o_ref[...] = a_ref[0] + b_ref[0] - n_ref[...]


def _combine(parts, node):
    row_blk = 2000
    return pl.pallas_call(
        _combine_body,
        grid=(N // row_blk,),
        in_specs=[
            pl.BlockSpec((1, row_blk, H), lambda i: (0, i, 0)),
            pl.BlockSpec((1, row_blk, H), lambda i: (1, i, 0)),
            pl.BlockSpec((row_blk, H), lambda i: (i, 0)),
        ],
        out_specs=pl.BlockSpec((row_blk, H), lambda i: (i, 0)),
        out_shape=jax.ShapeDtypeStruct((N, H), jnp.float32),
    )(parts, parts, node)


NUM_GRAPHS_OUT = 64
MLP_BLK = 2000


def _mlp_pool_body(pa_ref, pb_ref, n_ref, w1_ref, b1_ref, w2_ref, b2_ref,
                   w3_ref, batch_ref, o_ref):
    i = pl.program_id(0)
    h = pa_ref[0] + pb_ref[0] - n_ref[...]
    h = jnp.maximum(h, 0.0)
    h = jnp.dot(h, w1_ref[...], preferred_element_type=jnp.float32) + b1_ref[...]
    h = jnp.maximum(h, 0.0)
    h = jnp.dot(h, w2_ref[...], preferred_element_type=jnp.float32) + b2_ref[...]
    h = jnp.maximum(h, 0.0)
    e = jnp.dot(h, w3_ref[...], preferred_element_type=jnp.float32)
    b = batch_ref[...].reshape(MLP_BLK)
    ids = lax.broadcasted_iota(jnp.int32, (MLP_BLK, NUM_GRAPHS_OUT), 1)
    oh = (b[:, None] == ids).astype(jnp.float32)
    dgp = lax.dot_general(oh, e, (((0,), (0,)), ((), ())),
                          preferred_element_type=jnp.float32)

    @pl.when(i == 0)
    def _():
        o_ref[...] = jnp.zeros_like(o_ref)

    o_ref[...] += dgp


def _mlp_pool(parts, node, w1_t, b1, w2_t, b2, w3_t, batch3):
    grid = N // MLP_BLK
    return pl.pallas_call(
        _mlp_pool_body,
        grid=(grid,),
        in_specs=[
            pl.BlockSpec((1, MLP_BLK, H), lambda i: (0, i, 0)),
            pl.BlockSpec((1, MLP_BLK, H), lambda i: (1, i, 0)),
            pl.BlockSpec((MLP_BLK, H), lambda i: (i, 0)),
            pl.BlockSpec((H, H), lambda i: (0, 0)),
            pl.BlockSpec((1, H), lambda i: (0, 0)),
            pl.BlockSpec((H, H // 2), lambda i: (0, 0)),
            pl.BlockSpec((1, H // 2), lambda i: (0, 0)),
            pl.BlockSpec((H // 2, 1), lambda i: (0, 0)),
            pl.BlockSpec((1, 1, MLP_BLK), lambda i: (i, 0, 0)),
        ],
        out_specs=pl.BlockSpec((NUM_GRAPHS_OUT, 1), lambda i: (0, 0)),
        out_shape=jax.ShapeDtypeStruct((NUM_GRAPHS_OUT, 1), jnp.float32),
    )(parts, parts, node, w1_t, b1, w2_t, b2, w3_t, batch3)


def kernel(x, edge_index, edge_attr, batch, W_atom, b_atom, W_bond, b_bond,
           W1, b1, W2, b2, W3):
    src = edge_index[0].astype(jnp.int32)
    dst = edge_index[1].astype(jnp.int32)
    batch3 = batch.astype(jnp.int32).reshape(N // MLP_BLK, 1, MLP_BLK)

    node_emb = _matmul_bias(x, W_atom.T, b_atom.reshape(1, H), 2000)
    edge_emb = _matmul_bias(edge_attr, W_bond.T, b_bond.reshape(1, H), 4000)

    parts1 = _sc_layer(node_emb, edge_emb, src, dst)
    node1 = _combine(parts1, node_emb)
    parts2 = _sc_layer(node1, edge_emb, src, dst)

    dg = _mlp_pool(parts2, node1, W1.T, b1.reshape(1, H),
                   W2.T, b2.reshape(1, H // 2), W3.T, batch3)
    return dg


# speedup vs baseline: 1.0021x; 1.0021x over previous
import functools

import jax
import jax.numpy as jnp
from jax import lax
from jax.experimental import pallas as pl
from jax.experimental.pallas import tpu as pltpu
from jax.experimental.pallas import tpu_sc as plsc

NC = 2
NS = 16
LANES = 16

N = 10000
E = 320000
D = 128
H = 128

EPW = E // (NC * NS)
EB = 80
NBLK = EPW // EB
RPT = 624
TAIL = N - NS * RPT


def _sc_layer_body(node_hbm, emb_hbm, src_hbm, dst_hbm, out_hbm,
                   idx_s, idx_d, rows_v, emb_v, table_sh, sem,
                   sc_sem0, sc_sem1):
    c = lax.axis_index("c")
    s = lax.axis_index("s")
    wid = c * NS + s

    pltpu.sync_copy(node_hbm.at[pl.ds(s * RPT, RPT)],
                    table_sh.at[pl.ds(s * RPT, RPT)])

    @pl.when(s == 0)
    def _():
        pltpu.sync_copy(node_hbm.at[pl.ds(NS * RPT, TAIL)],
                        table_sh.at[pl.ds(NS * RPT, TAIL)])

    plsc.subcore_barrier()

    sc_sems = (sc_sem0, sc_sem1)

    def edge_block_slot(g, b):
        base = wid * EPW + g * EB

        def _wait_prev():
            pltpu.make_async_copy(rows_v.at[b], table_sh.at[idx_d.at[b]],
                                  sc_sems[b]).wait()

        if isinstance(g, int):
            if g >= 2:
                _wait_prev()
        else:
            pl.when(g >= 2)(_wait_prev)

        pltpu.sync_copy(src_hbm.at[pl.ds(base, EB)], idx_s.at[b])
        pltpu.sync_copy(dst_hbm.at[pl.ds(base, EB)], idx_d.at[b])
        gather = pltpu.async_copy(node_hbm.at[idx_s.at[b]], rows_v.at[b], sem)
        pltpu.sync_copy(emb_hbm.at[pl.ds(base, EB)], emb_v)
        gather.wait()

        def mul_row(i, carry2):
            for d in range(H // LANES):
                sl = pl.ds(d * LANES, LANES)
                rows_v[b, i, sl] = rows_v[b, i, sl] * emb_v[i, sl]
            return carry2

        lax.fori_loop(0, EB, mul_row, 0, unroll=False)
        pltpu.async_copy(rows_v.at[b], table_sh.at[idx_d.at[b]], sc_sems[b],
                         add=True)

    def edge_block(h, carry):
        edge_block_slot(2 * h, 0)
        edge_block_slot(2 * h + 1, 1)
        return carry

    lax.fori_loop(0, NBLK // 2, edge_block, 0, unroll=False)
    if NBLK % 2:
        edge_block_slot(NBLK - 1, 0)
    pltpu.make_async_copy(rows_v.at[0], table_sh.at[idx_d.at[0]],
                          sc_sems[0]).wait()
    pltpu.make_async_copy(rows_v.at[1], table_sh.at[idx_d.at[1]],
                          sc_sems[1]).wait()
    plsc.subcore_barrier()

    pltpu.sync_copy(table_sh.at[pl.ds(s * RPT, RPT)],
                    out_hbm.at[c, pl.ds(s * RPT, RPT)])

    @pl.when(s == 0)
    def _():
        pltpu.sync_copy(table_sh.at[pl.ds(NS * RPT, TAIL)],
                        out_hbm.at[c, pl.ds(NS * RPT, TAIL)])


@functools.cache
def _get_sc_layer():
    return pl.kernel(
        _sc_layer_body,
        out_type=jax.ShapeDtypeStruct((NC, N, H), jnp.float32),
        mesh=plsc.VectorSubcoreMesh(core_axis_name="c", subcore_axis_name="s",
                                    num_cores=NC, num_subcores=NS),
        scratch_types=[
            pltpu.VMEM((2, EB), jnp.int32),
            pltpu.VMEM((2, EB), jnp.int32),
            pltpu.VMEM((2, EB, H), jnp.float32),
            pltpu.VMEM((EB, H), jnp.float32),
            pltpu.VMEM_SHARED((N, H), jnp.float32),
            pltpu.SemaphoreType.DMA,
            pltpu.SemaphoreType.DMA,
            pltpu.SemaphoreType.DMA,
        ],
    )


def _sc_layer(*args):
    return _get_sc_layer()(*args)


def _matmul_bias_body(x_ref, w_ref, b_ref, o_ref):
    o_ref[...] = jnp.dot(x_ref[...], w_ref[...],
                         preferred_element_type=jnp.float32) + b_ref[...]


def _matmul_bias(x, w_t, b, row_blk):
    rows, k = x.shape
    cols = w_t.shape[1]
    grid = rows // row_blk
    return pl.pallas_call(
        _matmul_bias_body,
        grid=(grid,),
        in_specs=[
            pl.BlockSpec((row_blk, k), lambda i: (i, 0)),
            pl.BlockSpec((k, cols), lambda i: (0, 0)),
            pl.BlockSpec((1, cols), lambda i: (0, 0)),
        ],
        out_specs=pl.BlockSpec((row_blk, cols), lambda i: (i, 0)),
        out_shape=jax.ShapeDtypeStruct((rows, cols), jnp.float32),
    )(x, w_t, b)


def _combine_body(a_ref, b_ref, n_ref, o_ref):
    o_ref[...] = a_ref[0] + b_ref[0] - n_ref[...]


def _combine(parts, node):
    row_blk = 2000
    return pl.pallas_call(
        _combine_body,
        grid=(N // row_blk,),
        in_specs=[
            pl.BlockSpec((1, row_blk, H), lambda i: (0, i, 0)),
            pl.BlockSpec((1, row_blk, H), lambda i: (1, i, 0)),
            pl.BlockSpec((row_blk, H), lambda i: (i, 0)),
        ],
        out_specs=pl.BlockSpec((row_blk, H), lambda i: (i, 0)),
        out_shape=jax.ShapeDtypeStruct((N, H), jnp.float32),
    )(parts, parts, node)


NUM_GRAPHS_OUT = 64
MLP_BLK = 2000


def _mlp_pool_body(pa_ref, pb_ref, n_ref, w1_ref, b1_ref, w2_ref, b2_ref,
                   w3_ref, batch_ref, o_ref):
    i = pl.program_id(0)
    h = pa_ref[0] + pb_ref[0] - n_ref[...]
    h = jnp.maximum(h, 0.0)
    h = jnp.dot(h, w1_ref[...], preferred_element_type=jnp.float32) + b1_ref[...]
    h = jnp.maximum(h, 0.0)
    h = jnp.dot(h, w2_ref[...], preferred_element_type=jnp.float32) + b2_ref[...]
    h = jnp.maximum(h, 0.0)
    e = jnp.dot(h, w3_ref[...], preferred_element_type=jnp.float32)
    b = batch_ref[...].reshape(MLP_BLK)
    ids = lax.broadcasted_iota(jnp.int32, (MLP_BLK, NUM_GRAPHS_OUT), 1)
    oh = (b[:, None] == ids).astype(jnp.float32)
    dgp = lax.dot_general(oh, e, (((0,), (0,)), ((), ())),
                          preferred_element_type=jnp.float32)

    @pl.when(i == 0)
    def _():
        o_ref[...] = jnp.zeros_like(o_ref)

    o_ref[...] += dgp


def _mlp_pool(parts, node, w1_t, b1, w2_t, b2, w3_t, batch3):
    grid = N // MLP_BLK
    return pl.pallas_call(
        _mlp_pool_body,
        grid=(grid,),
        in_specs=[
            pl.BlockSpec((1, MLP_BLK, H), lambda i: (0, i, 0)),
            pl.BlockSpec((1, MLP_BLK, H), lambda i: (1, i, 0)),
            pl.BlockSpec((MLP_BLK, H), lambda i: (i, 0)),
            pl.BlockSpec((H, H), lambda i: (0, 0)),
            pl.BlockSpec((1, H), lambda i: (0, 0)),
            pl.BlockSpec((H, H // 2), lambda i: (0, 0)),
            pl.BlockSpec((1, H // 2), lambda i: (0, 0)),
            pl.BlockSpec((H // 2, 1), lambda i: (0, 0)),
            pl.BlockSpec((1, 1, MLP_BLK), lambda i: (i, 0, 0)),
        ],
        out_specs=pl.BlockSpec((NUM_GRAPHS_OUT, 1), lambda i: (0, 0)),
        out_shape=jax.ShapeDtypeStruct((NUM_GRAPHS_OUT, 1), jnp.float32),
    )(parts, parts, node, w1_t, b1, w2_t, b2, w3_t, batch3)


def kernel(x, edge_index, edge_attr, batch, W_atom, b_atom, W_bond, b_bond,
           W1, b1, W2, b2, W3):
    src = edge_index[0].astype(jnp.int32)
    dst = edge_index[1].astype(jnp.int32)
    batch3 = batch.astype(jnp.int32).reshape(N // MLP_BLK, 1, MLP_BLK)

    node_emb = _matmul_bias(x, W_atom.T, b_atom.reshape(1, H), 2000)
    edge_emb = _matmul_bias(edge_attr, W_bond.T, b_bond.reshape(1, H), 4000)

    parts1 = _sc_layer(node_emb, edge_emb, src, dst)
    node1 = _combine(parts1, node_emb)
    parts2 = _sc_layer(node1, edge_emb, src, dst)

    dg = _mlp_pool(parts2, node1, W1.T, b1.reshape(1, H),
                   W2.T, b2.reshape(1, H // 2), W3.T, batch3)
    return dg
